# baseline (speedup 1.0000x reference)
import functools

import jax
import jax.numpy as jnp
from jax import lax
from jax.experimental import pallas as pl
from jax.experimental.pallas import tpu as pltpu
from jax.experimental.pallas import tpu_sc as plsc

N = 10000
D = 128
DA = 144
D2 = 16
CHUNK = 128
NC = 2
NS = 16
NW = NC * NS
ROWS_ACC = 10240
RPT = ROWS_ACC // NS
RB = 2000


def _make_sc_agg(dw, e_pad):
    epw = e_pad // NW
    n_chunks = epw // CHUNK
    wide = dw > 64
    mesh = plsc.VectorSubcoreMesh(core_axis_name="c", subcore_axis_name="s")

    if wide:
        scratch = [
            pltpu.VMEM((CHUNK,), jnp.int32),
            pltpu.VMEM((CHUNK,), jnp.int32),
            pltpu.VMEM((CHUNK, dw), jnp.float32),
            pltpu.VMEM_SHARED((ROWS_ACC, dw), jnp.float32),
            pltpu.SemaphoreType.DMA,
        ]
    else:
        scratch = [
            pltpu.VMEM((CHUNK,), jnp.int32),
            pltpu.VMEM((CHUNK,), jnp.int32),
            pltpu.VMEM((CHUNK, dw), jnp.float32),
            pltpu.VMEM_SHARED((ROWS_ACC, dw), jnp.float32),
            pltpu.SemaphoreType.DMA,
        ] + 3 * [
            pltpu.VMEM((CHUNK,), jnp.int32),
            pltpu.VMEM((CHUNK,), jnp.int32),
            pltpu.VMEM((CHUNK, dw), jnp.float32),
            pltpu.SemaphoreType.DMA,
        ]

    @functools.partial(
        pl.kernel,
        out_type=jax.ShapeDtypeStruct((NC, ROWS_ACC, dw), jnp.float32),
        mesh=mesh,
        scratch_types=scratch,
        compiler_params=pltpu.CompilerParams(use_tc_tiling_on_sc=False),
    )
    def sc_agg(tbl_hbm, src_hbm, dst_hbm, out_hbm, src_v0, dst_v0,
               rows_a, acc_sh, sem_a, *extra):
        c = lax.axis_index("c")
        s = lax.axis_index("s")
        wid = s * NC + c

        zv = jnp.zeros((16,), jnp.float32)

        @pl.loop(0, CHUNK)
        def _(i):
            for j in range(dw // 16):
                rows_a[i, pl.ds(j * 16, 16)] = zv

        for k in range(RPT // CHUNK):
            pltpu.sync_copy(rows_a, acc_sh.at[pl.ds(s * RPT + k * CHUNK, CHUNK)])
        plsc.subcore_barrier()

        if wide:
            @pl.loop(0, n_chunks)
            def _(j):
                base = pl.multiple_of(wid * epw + j * CHUNK, 8)
                pltpu.sync_copy(src_hbm.at[pl.ds(base, CHUNK)], src_v0)
                pltpu.sync_copy(dst_hbm.at[pl.ds(base, CHUNK)], dst_v0)
                pltpu.async_copy(tbl_hbm.at[src_v0], rows_a, sem_a).wait()
                pltpu.sync_copy(rows_a, acc_sh.at[dst_v0], add=True)
        else:
            DEPTH = 4
            srcs = (src_v0,) + tuple(extra[0::4])
            dsts = (dst_v0,) + tuple(extra[1::4])
            rows = (rows_a,) + tuple(extra[2::4])
            sems = (sem_a,) + tuple(extra[3::4])

            @pl.loop(0, n_chunks // DEPTH)
            def _(b):
                ds = []
                for u in range(DEPTH):
                    base = pl.multiple_of(
                        wid * epw + (b * DEPTH + u) * CHUNK, 8)
                    pltpu.sync_copy(src_hbm.at[pl.ds(base, CHUNK)], srcs[u])
                    pltpu.sync_copy(dst_hbm.at[pl.ds(base, CHUNK)], dsts[u])
                    ds.append(pltpu.async_copy(tbl_hbm.at[srcs[u]], rows[u],
                                               sems[u]))
                for u in range(DEPTH):
                    ds[u].wait()
                    pltpu.sync_copy(rows[u], acc_sh.at[dsts[u]], add=True)

            for t in range(n_chunks % DEPTH):
                j = (n_chunks // DEPTH) * DEPTH + t
                base = pl.multiple_of(wid * epw + j * CHUNK, 8)
                pltpu.sync_copy(src_hbm.at[pl.ds(base, CHUNK)], src_v0)
                pltpu.sync_copy(dst_hbm.at[pl.ds(base, CHUNK)], dst_v0)
                pltpu.async_copy(tbl_hbm.at[src_v0], rows_a, sem_a).wait()
                pltpu.sync_copy(rows_a, acc_sh.at[dst_v0], add=True)

        plsc.subcore_barrier()
        pltpu.sync_copy(acc_sh.at[pl.ds(s * RPT, RPT)],
                        out_hbm.at[c, pl.ds(s * RPT, RPT)])

    return sc_agg


def _tc_pre_body(x_ref, w1r_ref, b1l_ref, w2l_ref, w2r_ref, wc_ref, bc_ref,
                 b2l_ref, xr_ref, ab_ref):
    xr_ref[...] = (jnp.dot(x_ref[...], w1r_ref[...],
                           preferred_element_type=jnp.float32) + b1l_ref[...])
    a = jnp.dot(w2l_ref[...], wc_ref[...], preferred_element_type=jnp.float32)
    b = jnp.dot(w2r_ref[...], wc_ref[...], preferred_element_type=jnp.float32)
    cb = (jnp.dot(b2l_ref[...], wc_ref[...], preferred_element_type=jnp.float32)
          + bc_ref[...])
    ab_ref[...] = jnp.concatenate(
        [a, b, jnp.broadcast_to(cb, (D, 2)), jnp.zeros((D, 2), jnp.float32)],
        axis=1)


def _tc_pre(x, w1r, b1l, w2l, w2r, wc, bc, b2l):
    full = lambda shp: pl.BlockSpec(shp, lambda i: tuple(0 for _ in shp))
    return pl.pallas_call(
        _tc_pre_body,
        grid=(N // RB,),
        in_specs=[
            pl.BlockSpec((RB, D), lambda i: (i, 0)),
            full((D, D)), full((1, D)), full((D, D)), full((D, D)),
            full((D, 2)), full((1, 2)), full((1, D)),
        ],
        out_specs=[
            pl.BlockSpec((RB, D), lambda i: (i, 0)),
            pl.BlockSpec((D, 8), lambda i: (0, 0)),
        ],
        out_shape=[
            jax.ShapeDtypeStruct((N, D), jnp.float32),
            jax.ShapeDtypeStruct((D, 8), jnp.float32),
        ],
    )(x, w1r, b1l, w2l, w2r, wc, bc, b2l)


def _tc_mid_body(p_ref, xr_ref, w1l_ref, ab_ref, o_ref):
    psum = p_ref[0] + p_ref[1]
    cnt = psum[:, D]
    inv = 1.0 / jnp.maximum(cnt, 1.0)
    mean = psum[:, :D] * inv[:, None]
    h = jnp.maximum(
        jnp.dot(mean, w1l_ref[...], preferred_element_type=jnp.float32)
        + xr_ref[...], 0.0)
    gr = jnp.dot(h, ab_ref[:, :4], preferred_element_type=jnp.float32)
    r = gr[:, 2:4] + ab_ref[0, 4:6][None, :]
    o_ref[...] = jnp.concatenate(
        [gr[:, :2], r, inv[:, None], jnp.zeros((RB, D2 - 5), jnp.float32)],
        axis=1)


def _tc_mid(p, xr, w1l, ab):
    full = lambda shp: pl.BlockSpec(shp, lambda i: tuple(0 for _ in shp))
    return pl.pallas_call(
        _tc_mid_body,
        grid=(N // RB,),
        in_specs=[
            pl.BlockSpec((NC, RB, DA), lambda i: (0, i, 0)),
            pl.BlockSpec((RB, D), lambda i: (i, 0)),
            full((D, D)), full((D, 8)),
        ],
        out_specs=pl.BlockSpec((RB, D2), lambda i: (i, 0)),
        out_shape=jax.ShapeDtypeStruct((N, D2), jnp.float32),
    )(p, xr, w1l, ab)


def _tc_out_body(q_ref, t_ref, o_ref):
    qsum = q_ref[0] + q_ref[1]
    o_ref[...] = qsum[:, :2] * t_ref[:, 4:5] + t_ref[:, 2:4]


def _tc_out(q, t):
    return pl.pallas_call(
        _tc_out_body,
        grid=(N // RB,),
        in_specs=[
            pl.BlockSpec((NC, RB, D2), lambda i: (0, i, 0)),
            pl.BlockSpec((RB, D2), lambda i: (i, 0)),
        ],
        out_specs=pl.BlockSpec((RB, 2), lambda i: (i, 0)),
        out_shape=jax.ShapeDtypeStruct((N, 2), jnp.float32),
    )(q, t)


def kernel(x, edge_index, W1_l, b1_l, W1_r, W2_l, b2_l, W2_r, Wc, bc):
    n, d = x.shape
    e = edge_index.shape[1]
    e_pad = -(-e // (NW * CHUNK)) * (NW * CHUNK)
    src = edge_index[0].astype(jnp.int32)
    dst = edge_index[1].astype(jnp.int32)
    if e_pad > e:
        src = jnp.concatenate([src, jnp.zeros((e_pad - e,), jnp.int32)])
        dst = jnp.concatenate([dst, jnp.full((e_pad - e,), n, jnp.int32)])

    x_aug = jnp.concatenate(
        [x, jnp.ones((n, 1), jnp.float32), jnp.zeros((n, DA - D - 1), jnp.float32)],
        axis=1)

    p1 = _make_sc_agg(DA, e_pad)(x_aug, src, dst)
    xr, ab = _tc_pre(x, W1_r, b1_l.reshape(1, D), W2_l, W2_r,
                     Wc, bc.reshape(1, 2), b2_l.reshape(1, D))
    t = _tc_mid(p1, xr, W1_l, ab)
    p2 = _make_sc_agg(D2, e_pad)(t, src, dst)
    return _tc_out(p2, t)


# speedup vs baseline: 1.0884x; 1.0884x over previous
import functools

import jax
import jax.numpy as jnp
from jax import lax
from jax.experimental import pallas as pl
from jax.experimental.pallas import tpu as pltpu
from jax.experimental.pallas import tpu_sc as plsc

N = 10000
D = 128
DA = 144
D2 = 16
CHUNK = 128
NC = 2
NS = 16
NW = NC * NS
ROWS_ACC = 10240
RPT = ROWS_ACC // NS
RB = 2000


def _make_sc_agg(dw, e_pad):
    epw = e_pad // NW
    n_chunks = epw // CHUNK
    wide = dw > 64
    mesh = plsc.VectorSubcoreMesh(core_axis_name="c", subcore_axis_name="s")

    if wide:
        scratch = [
            pltpu.VMEM((CHUNK,), jnp.int32),
            pltpu.VMEM((CHUNK,), jnp.int32),
            pltpu.VMEM((CHUNK, dw), jnp.float32),
            pltpu.VMEM_SHARED((ROWS_ACC, dw), jnp.float32),
            pltpu.SemaphoreType.DMA,
        ]
    else:
        scratch = [
            pltpu.VMEM((CHUNK,), jnp.int32),
            pltpu.VMEM((CHUNK,), jnp.int32),
            pltpu.VMEM((CHUNK, dw), jnp.float32),
            pltpu.VMEM_SHARED((ROWS_ACC, dw), jnp.float32),
            pltpu.SemaphoreType.DMA,
        ] + 5 * [
            pltpu.VMEM((CHUNK,), jnp.int32),
            pltpu.VMEM((CHUNK,), jnp.int32),
            pltpu.VMEM((CHUNK, dw), jnp.float32),
            pltpu.SemaphoreType.DMA,
        ]

    @functools.partial(
        pl.kernel,
        out_type=jax.ShapeDtypeStruct((NC, ROWS_ACC, dw), jnp.float32),
        mesh=mesh,
        scratch_types=scratch,
        compiler_params=pltpu.CompilerParams(use_tc_tiling_on_sc=False),
    )
    def sc_agg(tbl_hbm, src_hbm, dst_hbm, out_hbm, src_v0, dst_v0,
               rows_a, acc_sh, sem_a, *extra):
        c = lax.axis_index("c")
        s = lax.axis_index("s")
        wid = s * NC + c

        zv = jnp.zeros((16,), jnp.float32)

        @pl.loop(0, CHUNK)
        def _(i):
            for j in range(dw // 16):
                rows_a[i, pl.ds(j * 16, 16)] = zv

        for k in range(RPT // CHUNK):
            pltpu.sync_copy(rows_a, acc_sh.at[pl.ds(s * RPT + k * CHUNK, CHUNK)])
        plsc.subcore_barrier()

        if wide:
            @pl.loop(0, n_chunks)
            def _(j):
                base = pl.multiple_of(wid * epw + j * CHUNK, 8)
                pltpu.sync_copy(src_hbm.at[pl.ds(base, CHUNK)], src_v0)
                pltpu.sync_copy(dst_hbm.at[pl.ds(base, CHUNK)], dst_v0)
                pltpu.async_copy(tbl_hbm.at[src_v0], rows_a, sem_a).wait()
                pltpu.sync_copy(rows_a, acc_sh.at[dst_v0], add=True)
        else:
            DEPTH = 6
            srcs = (src_v0,) + tuple(extra[0::4])
            dsts = (dst_v0,) + tuple(extra[1::4])
            rows = (rows_a,) + tuple(extra[2::4])
            sems = (sem_a,) + tuple(extra[3::4])

            @pl.loop(0, n_chunks // DEPTH)
            def _(b):
                ds = []
                for u in range(DEPTH):
                    base = pl.multiple_of(
                        wid * epw + (b * DEPTH + u) * CHUNK, 8)
                    pltpu.sync_copy(src_hbm.at[pl.ds(base, CHUNK)], srcs[u])
                    pltpu.sync_copy(dst_hbm.at[pl.ds(base, CHUNK)], dsts[u])
                    ds.append(pltpu.async_copy(tbl_hbm.at[srcs[u]], rows[u],
                                               sems[u]))
                for u in range(DEPTH):
                    ds[u].wait()
                    pltpu.sync_copy(rows[u], acc_sh.at[dsts[u]], add=True)

            for t in range(n_chunks % DEPTH):
                j = (n_chunks // DEPTH) * DEPTH + t
                base = pl.multiple_of(wid * epw + j * CHUNK, 8)
                pltpu.sync_copy(src_hbm.at[pl.ds(base, CHUNK)], src_v0)
                pltpu.sync_copy(dst_hbm.at[pl.ds(base, CHUNK)], dst_v0)
                pltpu.async_copy(tbl_hbm.at[src_v0], rows_a, sem_a).wait()
                pltpu.sync_copy(rows_a, acc_sh.at[dst_v0], add=True)

        plsc.subcore_barrier()
        pltpu.sync_copy(acc_sh.at[pl.ds(s * RPT, RPT)],
                        out_hbm.at[c, pl.ds(s * RPT, RPT)])

    return sc_agg


def _tc_mid_body(p_ref, x_ref, w1l_ref, b1l_ref, w1r_ref, w2l_ref, b2l_ref,
                 w2r_ref, wc_ref, bc_ref, o_ref):
    psum = p_ref[0] + p_ref[1]
    cnt = psum[:, D]
    inv = 1.0 / jnp.maximum(cnt, 1.0)
    mean = psum[:, :D] * inv[:, None]
    h = jnp.maximum(
        jnp.dot(mean, w1l_ref[...], preferred_element_type=jnp.float32)
        + jnp.dot(x_ref[...], w1r_ref[...], preferred_element_type=jnp.float32)
        + b1l_ref[...], 0.0)
    a = jnp.dot(w2l_ref[...], wc_ref[...], preferred_element_type=jnp.float32)
    b = jnp.dot(w2r_ref[...], wc_ref[...], preferred_element_type=jnp.float32)
    g = jnp.dot(h, a, preferred_element_type=jnp.float32)
    r = (jnp.dot(h, b, preferred_element_type=jnp.float32)
         + jnp.dot(b2l_ref[...], wc_ref[...], preferred_element_type=jnp.float32)
         + bc_ref[...])
    o_ref[...] = jnp.concatenate(
        [g, r, inv[:, None], jnp.zeros((RB, D2 - 5), jnp.float32)], axis=1)


def _tc_mid(p, x, w1l, b1l, w1r, w2l, b2l, w2r, wc, bc):
    full = lambda shp: pl.BlockSpec(shp, lambda i: tuple(0 for _ in shp))
    return pl.pallas_call(
        _tc_mid_body,
        grid=(N // RB,),
        in_specs=[
            pl.BlockSpec((NC, RB, DA), lambda i: (0, i, 0)),
            pl.BlockSpec((RB, D), lambda i: (i, 0)),
            full((D, D)), full((1, D)), full((D, D)), full((D, D)),
            full((1, D)), full((D, D)), full((D, 2)), full((1, 2)),
        ],
        out_specs=pl.BlockSpec((RB, D2), lambda i: (i, 0)),
        out_shape=jax.ShapeDtypeStruct((N, D2), jnp.float32),
    )(p, x, w1l, b1l, w1r, w2l, b2l, w2r, wc, bc)


def _tc_out_body(q_ref, t_ref, o_ref):
    qsum = q_ref[0] + q_ref[1]
    o_ref[...] = qsum[:, :2] * t_ref[:, 4:5] + t_ref[:, 2:4]


def _tc_out(q, t):
    return pl.pallas_call(
        _tc_out_body,
        grid=(N // RB,),
        in_specs=[
            pl.BlockSpec((NC, RB, D2), lambda i: (0, i, 0)),
            pl.BlockSpec((RB, D2), lambda i: (i, 0)),
        ],
        out_specs=pl.BlockSpec((RB, 2), lambda i: (i, 0)),
        out_shape=jax.ShapeDtypeStruct((N, 2), jnp.float32),
    )(q, t)


def kernel(x, edge_index, W1_l, b1_l, W1_r, W2_l, b2_l, W2_r, Wc, bc):
    n, d = x.shape
    e = edge_index.shape[1]
    e_pad = -(-e // (NW * CHUNK)) * (NW * CHUNK)
    src = edge_index[0].astype(jnp.int32)
    dst = edge_index[1].astype(jnp.int32)
    if e_pad > e:
        src = jnp.concatenate([src, jnp.zeros((e_pad - e,), jnp.int32)])
        dst = jnp.concatenate([dst, jnp.full((e_pad - e,), n, jnp.int32)])

    x_aug = jnp.concatenate(
        [x, jnp.ones((n, 1), jnp.float32), jnp.zeros((n, DA - D - 1), jnp.float32)],
        axis=1)

    p1 = _make_sc_agg(DA, e_pad)(x_aug, src, dst)
    t = _tc_mid(p1, x,
                W1_l, b1_l.reshape(1, D), W1_r,
                W2_l, b2_l.reshape(1, D), W2_r,
                Wc, bc.reshape(1, 2))
    p2 = _make_sc_agg(D2, e_pad)(t, src, dst)
    return _tc_out(p2, t)
